# baseline (speedup 1.0000x reference)
import functools

import jax
import jax.numpy as jnp
from jax.experimental import pallas as pl
from jax.experimental.pallas import tpu as pltpu
from jax.experimental.pallas import tpu_sc as plsc

_E = 8
_EPAD = 128
_TBLK = 256
_NSUB = 16
_LANES = 16


def _router_body(x_ref, wg_ref, bg_ref, logits_ref):
    x = x_ref[...]
    lg = jnp.dot(x, wg_ref[...], preferred_element_type=jnp.float32)
    lg = lg + bg_ref[...]
    logits_ref[...] = lg[:, :_E]


def _sc_dispatch_body(lg_hbm, p1_hbm, p2_hbm, g1_hbm, g2_hbm, meta_hbm,
                      cx_hbm, lgv, i1v, i2v, w1v, w2v, lr1v, lr2v, basev,
                      callv):
    tpw = lgv.shape[0] // _E
    ngrp = tpw // _LANES
    wid = jax.lax.axis_index("s")
    base_t = wid * tpw
    pltpu.sync_copy(lg_hbm.at[pl.ds(base_t * _E, tpw * _E)], lgv)

    li = jax.lax.iota(jnp.int32, _LANES)
    neg = jnp.float32(-jnp.inf)
    cnt = [jnp.int32(0)] * _E
    for g in range(ngrp):
        idx0 = (li + g * _LANES) * _E
        ve = [plsc.load_gather(lgv, [idx0 + e]) for e in range(_E)]
        m1 = jnp.full((_LANES,), neg, jnp.float32)
        i1 = jnp.zeros((_LANES,), jnp.int32)
        for e in range(_E):
            gt = ve[e] > m1
            m1 = jnp.where(gt, ve[e], m1)
            i1 = jnp.where(gt, e, i1)
        m2 = jnp.full((_LANES,), neg, jnp.float32)
        i2 = jnp.zeros((_LANES,), jnp.int32)
        for e in range(_E):
            vm = jnp.where(i1 == e, neg, ve[e])
            gt = vm > m2
            m2 = jnp.where(gt, vm, m2)
            i2 = jnp.where(gt, e, i2)
        w1 = 1.0 / (1.0 + jnp.exp(m2 - m1))
        w2 = 1.0 - w1
        lr1 = jnp.zeros((_LANES,), jnp.int32)
        lr2 = jnp.zeros((_LANES,), jnp.int32)
        for e in range(_E):
            me = (i1 == e) | (i2 == e)
            mei = me.astype(jnp.int32)
            ex = plsc.cumsum(mei) - mei
            rk = cnt[e] + ex
            lr1 = jnp.where(i1 == e, rk, lr1)
            lr2 = jnp.where(i2 == e, rk, lr2)
            cnt[e] = cnt[e] + jnp.sum(mei)
        sl = pl.ds(g * _LANES, _LANES)
        i1v[sl] = i1
        i2v[sl] = i2
        w1v[sl] = w1
        w2v[sl] = w2
        lr1v[sl] = lr1
        lr2v[sl] = lr2

    cvec = jnp.zeros((_LANES,), jnp.int32)
    for e in range(_E):
        cvec = jnp.where(li == e, cnt[e], cvec)
    basev[...] = cvec
    pltpu.sync_copy(basev, cx_hbm.at[wid])
    plsc.subcore_barrier()
    pltpu.sync_copy(cx_hbm, callv)

    pref = jnp.zeros((_LANES,), jnp.int32)
    tot = jnp.zeros((_LANES,), jnp.int32)
    for j in range(_NSUB):
        rowj = callv[j]
        pref = pref + jnp.where(j < wid, rowj, jnp.zeros_like(rowj))
        tot = tot + rowj
    cnt_pad = jax.lax.shift_left(
        jax.lax.shift_right_logical(tot + (_TBLK - 1), 8), 8)
    pstart = plsc.cumsum(cnt_pad) - cnt_pad
    basev[...] = pstart + pref

    @pl.when(wid == 0)
    def _():
        callv[0] = pstart
        callv[1] = jax.lax.shift_right_logical(cnt_pad, 8)
        pltpu.sync_copy(callv.at[pl.ds(0, 2)], meta_hbm)

    for g in range(ngrp):
        sl = pl.ds(g * _LANES, _LANES)
        p1g = plsc.load_gather(basev, [i1v[sl]]) + lr1v[sl]
        p2g = plsc.load_gather(basev, [i2v[sl]]) + lr2v[sl]
        i1v[sl] = p1g
        i2v[sl] = p2g
    out_sl = pl.ds(base_t, tpw)
    pltpu.sync_copy(i1v, p1_hbm.at[out_sl])
    pltpu.sync_copy(i2v, p2_hbm.at[out_sl])
    pltpu.sync_copy(w1v, g1_hbm.at[out_sl])
    pltpu.sync_copy(w2v, g2_hbm.at[out_sl])


def _ffn_body(meta_ref, pdr_ref, pdc_ref, wdc_ref, xbf_ref,
              w1_ref, b1_ref, w2_ref, b2_ref, out_ref):
    e = pl.program_id(0)
    T, D = xbf_ref.shape
    base = meta_ref[e]
    nch = meta_ref[8 + e]
    b1r = b1_ref[0]
    b2r = b2_ref[0]
    p1r = pdr_ref[0]
    p2r = pdr_ref[1]
    p1c = pdc_ref[:, 0:1]
    p2c = pdc_ref[:, 1:2]
    g1c = wdc_ref[:, 0:1]
    g2c = wdc_ref[:, 1:2]
    xb = xbf_ref[...]

    @pl.when(e == 0)
    def _():
        out_ref[...] = jnp.zeros_like(out_ref)

    def chunk(c, carry):
        s0 = base + c * _TBLK
        rr = jax.lax.broadcasted_iota(jnp.int32, (_TBLK, T), 0) + s0
        og = ((p1r == rr) | (p2r == rr)).astype(jnp.bfloat16)
        rc = jax.lax.broadcasted_iota(jnp.int32, (T, _TBLK), 1) + s0
        ogt = (jnp.where(p1c == rc, g1c, 0.0)
               + jnp.where(p2c == rc, g2c, 0.0)).astype(jnp.bfloat16)

        xg = jnp.dot(og, xb, preferred_element_type=jnp.float32)
        h = jnp.dot(xg, w1_ref[0], preferred_element_type=jnp.float32) + b1r
        h = jax.nn.gelu(h)
        y = jnp.dot(h, w2_ref[0], preferred_element_type=jnp.float32) + b2r
        out_ref[...] += jnp.dot(ogt, y.astype(jnp.bfloat16),
                                preferred_element_type=jnp.float32)
        return carry

    jax.lax.fori_loop(0, nch, chunk, 0)


def kernel(x, Wg, bg, W1, b1, W2, b2):
    Bs, Ls, Ds = x.shape
    T = Bs * Ls
    E, Dff = W1.shape[0], W1.shape[2]
    x_flat = x.reshape(T, Ds)

    wg_pad = jnp.zeros((Ds, _EPAD), Wg.dtype).at[:, :E].set(Wg)
    bg_pad = jnp.full((1, _EPAD), -jnp.inf, bg.dtype).at[0, :E].set(bg)

    logits = pl.pallas_call(
        _router_body,
        out_shape=jax.ShapeDtypeStruct((T, E), jnp.float32),
    )(x_flat, wg_pad, bg_pad)
    xbf = x_flat.astype(jnp.bfloat16)

    tpw = T // _NSUB
    sc_dispatch = functools.partial(
        pl.kernel,
        out_type=[
            jax.ShapeDtypeStruct((T,), jnp.int32),
            jax.ShapeDtypeStruct((T,), jnp.int32),
            jax.ShapeDtypeStruct((T,), jnp.float32),
            jax.ShapeDtypeStruct((T,), jnp.float32),
            jax.ShapeDtypeStruct((2, _LANES), jnp.int32),
            jax.ShapeDtypeStruct((_NSUB, _LANES), jnp.int32),
        ],
        mesh=plsc.VectorSubcoreMesh(
            core_axis_name="c", subcore_axis_name="s", num_cores=1),
        compiler_params=pltpu.CompilerParams(needs_layout_passes=False),
        scratch_types=[
            pltpu.VMEM((tpw * _E,), jnp.float32),
            pltpu.VMEM((tpw,), jnp.int32),
            pltpu.VMEM((tpw,), jnp.int32),
            pltpu.VMEM((tpw,), jnp.float32),
            pltpu.VMEM((tpw,), jnp.float32),
            pltpu.VMEM((tpw,), jnp.int32),
            pltpu.VMEM((tpw,), jnp.int32),
            pltpu.VMEM((_LANES,), jnp.int32),
            pltpu.VMEM((_NSUB, _LANES), jnp.int32),
        ],
    )(_sc_dispatch_body)
    p1, p2, g1, g2, meta, _ = sc_dispatch(
        jnp.reshape(logits, (T * E,)))

    pdc = jnp.stack([p1, p2], axis=1)
    pdr = pdc.T.reshape(2, 1, T)
    wdc = jnp.stack([g1, g2], axis=1)
    meta_smem = jnp.concatenate([meta[0, :E], meta[1, :E]])

    out = pl.pallas_call(
        _ffn_body,
        grid=(E,),
        in_specs=[
            pl.BlockSpec(memory_space=pltpu.SMEM),
            pl.BlockSpec((2, 1, T), lambda e: (0, 0, 0)),
            pl.BlockSpec((T, 2), lambda e: (0, 0)),
            pl.BlockSpec((T, 2), lambda e: (0, 0)),
            pl.BlockSpec((T, Ds), lambda e: (0, 0)),
            pl.BlockSpec((1, Ds, Dff), lambda e: (e, 0, 0)),
            pl.BlockSpec((1, 1, Dff), lambda e: (e, 0, 0)),
            pl.BlockSpec((1, Dff, Ds), lambda e: (e, 0, 0)),
            pl.BlockSpec((1, 1, Ds), lambda e: (e, 0, 0)),
        ],
        out_specs=pl.BlockSpec((T, Ds), lambda e: (0, 0)),
        out_shape=jax.ShapeDtypeStruct((T, Ds), jnp.float32),
        compiler_params=pltpu.CompilerParams(
            dimension_semantics=("arbitrary",),
            vmem_limit_bytes=63 * 1024 * 1024,
        ),
    )(meta_smem, pdr, pdc, wdc, xbf,
      W1, b1.reshape(E, 1, Dff), W2, b2.reshape(E, 1, Ds))

    return out.reshape(Bs, Ls, Ds), logits


# speedup vs baseline: 1.0209x; 1.0209x over previous
import functools

import jax
import jax.numpy as jnp
from jax.experimental import pallas as pl
from jax.experimental.pallas import tpu as pltpu
from jax.experimental.pallas import tpu_sc as plsc

_E = 8
_EPAD = 128
_TBLK = 256
_NSUB = 16
_LANES = 16


def _router_body(x_ref, wg_ref, bg_ref, logits_ref, xbf_ref):
    x = x_ref[...]
    lg = jnp.dot(x, wg_ref[...], preferred_element_type=jnp.float32)
    lg = lg + bg_ref[...]
    logits_ref[...] = lg[:, :_E]
    xbf_ref[...] = x.astype(jnp.bfloat16)


def _sc_dispatch_body(lg_hbm, p1_hbm, p2_hbm, g1_hbm, g2_hbm, meta_hbm,
                      cx_hbm, lgv, i1v, i2v, w1v, w2v, lr1v, lr2v, basev,
                      callv):
    tpw = lgv.shape[0] // _E
    ngrp = tpw // _LANES
    wid = jax.lax.axis_index("s")
    base_t = wid * tpw
    pltpu.sync_copy(lg_hbm.at[pl.ds(base_t * _E, tpw * _E)], lgv)

    li = jax.lax.iota(jnp.int32, _LANES)
    neg = jnp.float32(-jnp.inf)
    cnt = [jnp.int32(0)] * _E
    for g in range(ngrp):
        idx0 = (li + g * _LANES) * _E
        ve = [plsc.load_gather(lgv, [idx0 + e]) for e in range(_E)]
        m1 = jnp.full((_LANES,), neg, jnp.float32)
        i1 = jnp.zeros((_LANES,), jnp.int32)
        for e in range(_E):
            gt = ve[e] > m1
            m1 = jnp.where(gt, ve[e], m1)
            i1 = jnp.where(gt, e, i1)
        m2 = jnp.full((_LANES,), neg, jnp.float32)
        i2 = jnp.zeros((_LANES,), jnp.int32)
        for e in range(_E):
            vm = jnp.where(i1 == e, neg, ve[e])
            gt = vm > m2
            m2 = jnp.where(gt, vm, m2)
            i2 = jnp.where(gt, e, i2)
        w1 = 1.0 / (1.0 + jnp.exp(m2 - m1))
        w2 = 1.0 - w1
        lr1 = jnp.zeros((_LANES,), jnp.int32)
        lr2 = jnp.zeros((_LANES,), jnp.int32)
        for e in range(_E):
            me = (i1 == e) | (i2 == e)
            mei = me.astype(jnp.int32)
            ex = plsc.cumsum(mei) - mei
            rk = cnt[e] + ex
            lr1 = jnp.where(i1 == e, rk, lr1)
            lr2 = jnp.where(i2 == e, rk, lr2)
            cnt[e] = cnt[e] + jnp.sum(mei)
        sl = pl.ds(g * _LANES, _LANES)
        i1v[sl] = i1
        i2v[sl] = i2
        w1v[sl] = w1
        w2v[sl] = w2
        lr1v[sl] = lr1
        lr2v[sl] = lr2

    cvec = jnp.zeros((_LANES,), jnp.int32)
    for e in range(_E):
        cvec = jnp.where(li == e, cnt[e], cvec)
    basev[...] = cvec
    pltpu.sync_copy(basev, cx_hbm.at[wid])
    plsc.subcore_barrier()
    pltpu.sync_copy(cx_hbm, callv)

    pref = jnp.zeros((_LANES,), jnp.int32)
    tot = jnp.zeros((_LANES,), jnp.int32)
    for j in range(_NSUB):
        rowj = callv[j]
        pref = pref + jnp.where(j < wid, rowj, jnp.zeros_like(rowj))
        tot = tot + rowj
    cnt_pad = jax.lax.shift_left(
        jax.lax.shift_right_logical(tot + (_TBLK - 1), 8), 8)
    pstart = plsc.cumsum(cnt_pad) - cnt_pad
    basev[...] = pstart + pref

    @pl.when(wid == 0)
    def _():
        callv[0] = pstart
        callv[1] = jax.lax.shift_right_logical(cnt_pad, 8)
        pltpu.sync_copy(callv.at[pl.ds(0, 2)], meta_hbm)

    for g in range(ngrp):
        sl = pl.ds(g * _LANES, _LANES)
        p1g = plsc.load_gather(basev, [i1v[sl]]) + lr1v[sl]
        p2g = plsc.load_gather(basev, [i2v[sl]]) + lr2v[sl]
        i1v[sl] = p1g
        i2v[sl] = p2g
    out_sl = pl.ds(base_t, tpw)
    pltpu.sync_copy(i1v, p1_hbm.at[out_sl])
    pltpu.sync_copy(i2v, p2_hbm.at[out_sl])
    pltpu.sync_copy(w1v, g1_hbm.at[out_sl])
    pltpu.sync_copy(w2v, g2_hbm.at[out_sl])


def _ffn_body(meta_ref, pdr_ref, pdc_ref, wdc_ref, xbf_ref,
              w1_ref, b1_ref, w2_ref, b2_ref, out_ref):
    e = pl.program_id(0)
    T, D = xbf_ref.shape
    base = meta_ref[e]
    nch = meta_ref[8 + e]
    b1r = b1_ref[0]
    b2r = b2_ref[0]
    p1r = pdr_ref[0]
    p2r = pdr_ref[1]
    p1c = pdc_ref[:, 0:1]
    p2c = pdc_ref[:, 1:2]
    g1c = wdc_ref[:, 0:1]
    g2c = wdc_ref[:, 1:2]
    xb = xbf_ref[...]

    @pl.when(e == 0)
    def _():
        out_ref[...] = jnp.zeros_like(out_ref)

    def chunk(c, carry):
        s0 = base + c * _TBLK
        rr = jax.lax.broadcasted_iota(jnp.int32, (_TBLK, T), 0) + s0
        og = ((p1r == rr) | (p2r == rr)).astype(jnp.bfloat16)
        rc = jax.lax.broadcasted_iota(jnp.int32, (T, _TBLK), 1) + s0
        ogt = (jnp.where(p1c == rc, g1c, 0.0)
               + jnp.where(p2c == rc, g2c, 0.0)).astype(jnp.bfloat16)

        xg = jnp.dot(og, xb, preferred_element_type=jnp.float32)
        h = jnp.dot(xg, w1_ref[0], preferred_element_type=jnp.float32) + b1r
        h = jax.nn.gelu(h)
        y = jnp.dot(h, w2_ref[0], preferred_element_type=jnp.float32) + b2r
        out_ref[...] += jnp.dot(ogt, y.astype(jnp.bfloat16),
                                preferred_element_type=jnp.float32)
        return carry

    jax.lax.fori_loop(0, nch, chunk, 0)


def kernel(x, Wg, bg, W1, b1, W2, b2):
    Bs, Ls, Ds = x.shape
    T = Bs * Ls
    E, Dff = W1.shape[0], W1.shape[2]
    x_flat = x.reshape(T, Ds)

    wg_pad = jnp.zeros((Ds, _EPAD), Wg.dtype).at[:, :E].set(Wg)
    bg_pad = jnp.full((1, _EPAD), -jnp.inf, bg.dtype).at[0, :E].set(bg)

    logits, xbf = pl.pallas_call(
        _router_body,
        out_shape=(
            jax.ShapeDtypeStruct((T, E), jnp.float32),
            jax.ShapeDtypeStruct((T, Ds), jnp.bfloat16),
        ),
    )(x_flat, wg_pad, bg_pad)

    tpw = T // _NSUB
    sc_dispatch = functools.partial(
        pl.kernel,
        out_type=[
            jax.ShapeDtypeStruct((T,), jnp.int32),
            jax.ShapeDtypeStruct((T,), jnp.int32),
            jax.ShapeDtypeStruct((T,), jnp.float32),
            jax.ShapeDtypeStruct((T,), jnp.float32),
            jax.ShapeDtypeStruct((2, _LANES), jnp.int32),
            jax.ShapeDtypeStruct((_NSUB, _LANES), jnp.int32),
        ],
        mesh=plsc.VectorSubcoreMesh(
            core_axis_name="c", subcore_axis_name="s", num_cores=1),
        compiler_params=pltpu.CompilerParams(needs_layout_passes=False),
        scratch_types=[
            pltpu.VMEM((tpw * _E,), jnp.float32),
            pltpu.VMEM((tpw,), jnp.int32),
            pltpu.VMEM((tpw,), jnp.int32),
            pltpu.VMEM((tpw,), jnp.float32),
            pltpu.VMEM((tpw,), jnp.float32),
            pltpu.VMEM((tpw,), jnp.int32),
            pltpu.VMEM((tpw,), jnp.int32),
            pltpu.VMEM((_LANES,), jnp.int32),
            pltpu.VMEM((_NSUB, _LANES), jnp.int32),
        ],
    )(_sc_dispatch_body)
    p1, p2, g1, g2, meta, _ = sc_dispatch(
        jnp.reshape(logits, (T * E,)))

    pdc = jnp.stack([p1, p2], axis=1)
    pdr = pdc.T.reshape(2, 1, T)
    wdc = jnp.stack([g1, g2], axis=1)
    meta_smem = jnp.concatenate([meta[0, :E], meta[1, :E]])

    out = pl.pallas_call(
        _ffn_body,
        grid=(E,),
        in_specs=[
            pl.BlockSpec(memory_space=pltpu.SMEM),
            pl.BlockSpec((2, 1, T), lambda e: (0, 0, 0)),
            pl.BlockSpec((T, 2), lambda e: (0, 0)),
            pl.BlockSpec((T, 2), lambda e: (0, 0)),
            pl.BlockSpec((T, Ds), lambda e: (0, 0)),
            pl.BlockSpec((1, Ds, Dff), lambda e: (e, 0, 0)),
            pl.BlockSpec((1, 1, Dff), lambda e: (e, 0, 0)),
            pl.BlockSpec((1, Dff, Ds), lambda e: (e, 0, 0)),
            pl.BlockSpec((1, 1, Ds), lambda e: (e, 0, 0)),
        ],
        out_specs=pl.BlockSpec((T, Ds), lambda e: (0, 0)),
        out_shape=jax.ShapeDtypeStruct((T, Ds), jnp.float32),
        compiler_params=pltpu.CompilerParams(
            dimension_semantics=("arbitrary",),
            vmem_limit_bytes=63 * 1024 * 1024,
        ),
    )(meta_smem, pdr, pdc, wdc, xbf,
      W1, b1.reshape(E, 1, Dff), W2, b2.reshape(E, 1, Ds))

    return out.reshape(Bs, Ls, Ds), logits
